# Initial kernel scaffold; baseline (speedup 1.0000x reference)
import functools

import jax
import jax.numpy as jnp
from jax import lax
from jax.experimental import pallas as pl
from jax.experimental.pallas import tpu as pltpu
from jax.experimental.pallas import tpu_sc as plsc

N = 10000
E = 320000
D = 128

NC = 2
NS = 16
K = 80
E_PER_TILE = E // (NC * NS)
NBATCH = E_PER_TILE // K
ROWS_PER_TILE = N // NS

_MESH = plsc.VectorSubcoreMesh(core_axis_name="c", subcore_axis_name="s")


def _sc_agg_body(with_counts, *refs):
    if with_counts:
        (x_hbm, src_hbm, dst_hbm, z_hbm, z16_hbm, ones_hbm,
         acc_out, cnt_out,
         src_v, dst_v, rows_v, ones_v, stage_v, stage16_v,
         acc_sh, cnt_sh, sem) = refs
    else:
        (x_hbm, src_hbm, dst_hbm, z_hbm,
         acc_out,
         src_v, dst_v, rows_v, stage_v,
         acc_sh, sem) = refs

    c = lax.axis_index("c")
    s = lax.axis_index("s")
    row0 = s * ROWS_PER_TILE

    pltpu.sync_copy(z_hbm, stage_v)
    pltpu.sync_copy(stage_v, acc_sh.at[pl.ds(row0, ROWS_PER_TILE)])
    if with_counts:
        pltpu.sync_copy(z16_hbm, stage16_v)
        pltpu.sync_copy(stage16_v, cnt_sh.at[pl.ds(row0, ROWS_PER_TILE)])
        pltpu.sync_copy(ones_hbm, ones_v)
    plsc.subcore_barrier()

    ebase = (c * NS + s) * E_PER_TILE

    def batch(i, carry):
        off = ebase + i * K
        pltpu.sync_copy(src_hbm.at[pl.ds(off, K)], src_v)
        pltpu.sync_copy(dst_hbm.at[pl.ds(off, K)], dst_v)
        pltpu.async_copy(x_hbm.at[src_v], rows_v, sem).wait()
        pltpu.sync_copy(rows_v, acc_sh.at[dst_v], add=True)
        if with_counts:
            pltpu.sync_copy(ones_v, cnt_sh.at[dst_v], add=True)
        return carry

    lax.fori_loop(0, NBATCH, batch, 0)
    plsc.subcore_barrier()

    pltpu.sync_copy(acc_sh.at[pl.ds(row0, ROWS_PER_TILE)], stage_v)
    pltpu.sync_copy(stage_v, acc_out.at[c, pl.ds(row0, ROWS_PER_TILE)])
    if with_counts:
        pltpu.sync_copy(cnt_sh.at[pl.ds(row0, ROWS_PER_TILE)], stage16_v)
        pltpu.sync_copy(stage16_v, cnt_out.at[c, pl.ds(row0, ROWS_PER_TILE)])


_sc_agg_counts = pl.kernel(
    functools.partial(_sc_agg_body, True),
    out_type=(
        jax.ShapeDtypeStruct((NC, N, D), jnp.float32),
        jax.ShapeDtypeStruct((NC, N, 16), jnp.float32),
    ),
    mesh=_MESH,
    scratch_types=[
        pltpu.VMEM((K,), jnp.int32),
        pltpu.VMEM((K,), jnp.int32),
        pltpu.VMEM((K, D), jnp.float32),
        pltpu.VMEM((K, 16), jnp.float32),
        pltpu.VMEM((ROWS_PER_TILE, D), jnp.float32),
        pltpu.VMEM((ROWS_PER_TILE, 16), jnp.float32),
        pltpu.VMEM_SHARED((N, D), jnp.float32),
        pltpu.VMEM_SHARED((N, 16), jnp.float32),
        pltpu.SemaphoreType.DMA,
    ],
    name="sage_sc_agg_counts",
)

_sc_agg = pl.kernel(
    functools.partial(_sc_agg_body, False),
    out_type=jax.ShapeDtypeStruct((NC, N, D), jnp.float32),
    mesh=_MESH,
    scratch_types=[
        pltpu.VMEM((K,), jnp.int32),
        pltpu.VMEM((K,), jnp.int32),
        pltpu.VMEM((K, D), jnp.float32),
        pltpu.VMEM((ROWS_PER_TILE, D), jnp.float32),
        pltpu.VMEM_SHARED((N, D), jnp.float32),
        pltpu.SemaphoreType.DMA,
    ],
    name="sage_sc_agg",
)


BLK = 1000


def _tc_dense_body(relu, acc_ref, cnt_ref, x_ref, wl_ref, wr_ref, b_ref,
                   out_ref):
    cnt = cnt_ref[0, :, 0:1] + cnt_ref[1, :, 0:1]
    mean = (acc_ref[0] + acc_ref[1]) / jnp.maximum(cnt, 1.0)
    r = jnp.dot(mean, wl_ref[...], preferred_element_type=jnp.float32)
    r = r + b_ref[...]
    r = r + jnp.dot(x_ref[...], wr_ref[...], preferred_element_type=jnp.float32)
    if relu:
        r = jnp.maximum(r, 0.0)
    out_ref[...] = r


def _tc_dense(accp, cntp, x, wl_t, wr_t, b, relu):
    return pl.pallas_call(
        functools.partial(_tc_dense_body, relu),
        grid=(N // BLK,),
        in_specs=[
            pl.BlockSpec((NC, BLK, D), lambda i: (0, i, 0)),
            pl.BlockSpec((NC, BLK, 16), lambda i: (0, i, 0)),
            pl.BlockSpec((BLK, D), lambda i: (i, 0)),
            pl.BlockSpec((D, D), lambda i: (0, 0)),
            pl.BlockSpec((D, D), lambda i: (0, 0)),
            pl.BlockSpec((1, D), lambda i: (0, 0)),
        ],
        out_specs=pl.BlockSpec((BLK, D), lambda i: (i, 0)),
        out_shape=jax.ShapeDtypeStruct((N, D), jnp.float32),
    )(accp, cntp, x, wl_t, wr_t, b)


def kernel(x, edge_index, W_l1, b_l1, W_r1, W_l2, b_l2, W_r2):
    src = edge_index[0]
    dst = edge_index[1]
    zeros_d = jnp.zeros((ROWS_PER_TILE, D), jnp.float32)
    zeros_16 = jnp.zeros((ROWS_PER_TILE, 16), jnp.float32)
    ones_k = jnp.ones((K, 16), jnp.float32)

    acc1, cnt = _sc_agg_counts(x, src, dst, zeros_d, zeros_16, ones_k)
    h = _tc_dense(acc1, cnt, x, W_l1.T, W_r1.T, b_l1[None, :], relu=True)
    acc2 = _sc_agg(h, src, dst, zeros_d)
    out = _tc_dense(acc2, cnt, h, W_l2.T, W_r2.T, b_l2[None, :], relu=False)
    return out


# speedup vs baseline: 5.4613x; 5.4613x over previous
import functools

import jax
import jax.numpy as jnp
from jax import lax
from jax.experimental import pallas as pl
from jax.experimental.pallas import tpu as pltpu
from jax.experimental.pallas import tpu_sc as plsc

N = 10000
E = 320000
D = 128

NC = 2
NS = 16
K = 80
E_PER_TILE = E // (NC * NS)
NBATCH = E_PER_TILE // K
N_PAD = 10240
ROWS_PER_TILE = N_PAD // NS

_MESH = plsc.VectorSubcoreMesh(core_axis_name="c", subcore_axis_name="s")


NCHUNK = ROWS_PER_TILE // K


def _sc_agg_body(with_counts, *refs):
    if with_counts:
        (x_hbm, src_hbm, dst_hbm, z_hbm, z16_hbm, ones_hbm,
         acc_out, cnt_out,
         src_v, dst_v, rows_v, ones_v, z16_v,
         acc_sh, cnt_sh, sem) = refs
    else:
        (x_hbm, src_hbm, dst_hbm, z_hbm,
         acc_out,
         src_v, dst_v, rows_v,
         acc_sh, sem) = refs

    c = lax.axis_index("c")
    s = lax.axis_index("s")
    row0 = s * ROWS_PER_TILE

    pltpu.sync_copy(z_hbm, rows_v)
    for j in range(NCHUNK):
        pltpu.sync_copy(rows_v, acc_sh.at[pl.ds(row0 + j * K, K)])
    if with_counts:
        pltpu.sync_copy(z16_hbm, z16_v)
        for j in range(NCHUNK):
            pltpu.sync_copy(z16_v, cnt_sh.at[pl.ds(row0 + j * K, K)])
        pltpu.sync_copy(ones_hbm, ones_v)
    plsc.subcore_barrier()

    ebase = (c * NS + s) * E_PER_TILE

    def batch(i, carry):
        off = ebase + i * K
        pltpu.sync_copy(src_hbm.at[pl.ds(off, K)], src_v)
        pltpu.sync_copy(dst_hbm.at[pl.ds(off, K)], dst_v)
        pltpu.async_copy(x_hbm.at[src_v], rows_v, sem).wait()
        pltpu.sync_copy(rows_v, acc_sh.at[dst_v], add=True)
        if with_counts:
            pltpu.sync_copy(ones_v, cnt_sh.at[dst_v], add=True)
        return carry

    lax.fori_loop(0, NBATCH, batch, 0)
    plsc.subcore_barrier()

    for j in range(NCHUNK):
        pltpu.sync_copy(acc_sh.at[pl.ds(row0 + j * K, K)], rows_v)
        pltpu.sync_copy(rows_v, acc_out.at[c, pl.ds(row0 + j * K, K)])
    if with_counts:
        for j in range(NCHUNK):
            pltpu.sync_copy(cnt_sh.at[pl.ds(row0 + j * K, K)], z16_v)
            pltpu.sync_copy(z16_v, cnt_out.at[c, pl.ds(row0 + j * K, K)])


_sc_agg_counts = pl.kernel(
    functools.partial(_sc_agg_body, True),
    out_type=(
        jax.ShapeDtypeStruct((NC, N_PAD, D), jnp.float32),
        jax.ShapeDtypeStruct((NC, N_PAD, 16), jnp.float32),
    ),
    mesh=_MESH,
    scratch_types=[
        pltpu.VMEM((K,), jnp.int32),
        pltpu.VMEM((K,), jnp.int32),
        pltpu.VMEM((K, D), jnp.float32),
        pltpu.VMEM((K, 16), jnp.float32),
        pltpu.VMEM((K, 16), jnp.float32),
        pltpu.VMEM_SHARED((N_PAD, D), jnp.float32),
        pltpu.VMEM_SHARED((N_PAD, 16), jnp.float32),
        pltpu.SemaphoreType.DMA,
    ],
    compiler_params=pltpu.CompilerParams(use_tc_tiling_on_sc=False),
    name="sage_sc_agg_counts",
)

_sc_agg = pl.kernel(
    functools.partial(_sc_agg_body, False),
    out_type=jax.ShapeDtypeStruct((NC, N_PAD, D), jnp.float32),
    mesh=_MESH,
    scratch_types=[
        pltpu.VMEM((K,), jnp.int32),
        pltpu.VMEM((K,), jnp.int32),
        pltpu.VMEM((K, D), jnp.float32),
        pltpu.VMEM_SHARED((N_PAD, D), jnp.float32),
        pltpu.SemaphoreType.DMA,
    ],
    compiler_params=pltpu.CompilerParams(use_tc_tiling_on_sc=False),
    name="sage_sc_agg",
)


BLK = 1000


def _tc_dense_body(relu, acc_ref, cnt_ref, x_ref, wl_ref, wr_ref, b_ref,
                   out_ref):
    cnt = cnt_ref[0, :, 0:1] + cnt_ref[1, :, 0:1]
    mean = (acc_ref[0] + acc_ref[1]) / jnp.maximum(cnt, 1.0)
    r = jnp.dot(mean, wl_ref[...], preferred_element_type=jnp.float32)
    r = r + b_ref[...]
    r = r + jnp.dot(x_ref[...], wr_ref[...], preferred_element_type=jnp.float32)
    if relu:
        r = jnp.maximum(r, 0.0)
    out_ref[...] = r


def _tc_dense(accp, cntp, x, wl_t, wr_t, b, relu):
    return pl.pallas_call(
        functools.partial(_tc_dense_body, relu),
        grid=(N // BLK,),
        in_specs=[
            pl.BlockSpec((NC, BLK, D), lambda i: (0, i, 0)),
            pl.BlockSpec((NC, BLK, 16), lambda i: (0, i, 0)),
            pl.BlockSpec((BLK, D), lambda i: (i, 0)),
            pl.BlockSpec((D, D), lambda i: (0, 0)),
            pl.BlockSpec((D, D), lambda i: (0, 0)),
            pl.BlockSpec((1, D), lambda i: (0, 0)),
        ],
        out_specs=pl.BlockSpec((BLK, D), lambda i: (i, 0)),
        out_shape=jax.ShapeDtypeStruct((N, D), jnp.float32),
    )(accp, cntp, x, wl_t, wr_t, b)


def kernel(x, edge_index, W_l1, b_l1, W_r1, W_l2, b_l2, W_r2):
    src = edge_index[0]
    dst = edge_index[1]
    zeros_d = jnp.zeros((K, D), jnp.float32)
    zeros_16 = jnp.zeros((K, 16), jnp.float32)
    ones_k = jnp.ones((K, 16), jnp.float32)

    acc1, cnt = _sc_agg_counts(x, src, dst, zeros_d, zeros_16, ones_k)
    h = _tc_dense(acc1, cnt, x, W_l1.T, W_r1.T, b_l1[None, :], relu=True)
    acc2 = _sc_agg(h, src, dst, zeros_d)
    out = _tc_dense(acc2, cnt, h, W_l2.T, W_r2.T, b_l2[None, :], relu=False)
    return out
